# baseline (speedup 1.0000x reference)
import functools

import jax
import jax.numpy as jnp
from jax import lax
from jax.experimental import pallas as pl
from jax.experimental.pallas import tpu as pltpu
from jax.experimental.pallas import tpu_sc as plsc

VOCAB = 100000
EMBED_DIM = 128
BATCH = 4096
HIST = 50

L_PAD = 64
NC, NS = 1, 16
NPAIR = 8
L_PER_CORE = L_PAD // NC
GRP = L_PER_CORE // 16
R_PER_SUB = BATCH // NPAIR
UNROLL = 8

MV_BLK = 8192
MV_GRID = (VOCAB + MV_BLK - 1) // MV_BLK
P_ROWS = MV_GRID * (MV_BLK // 128)
P_FLAT = P_ROWS * 128
P_HALF = P_FLAT // 2


def _mv_body(w_ref, e_ref, o_ref):
    e = e_ref[...]
    w = w_ref[...][0]
    prod = e.reshape(MV_BLK // 128, 128, 128) * w
    o_ref[...] = jnp.sum(prod, axis=-1)


def _matvec(emb_table, W):
    return pl.pallas_call(
        _mv_body,
        grid=(MV_GRID,),
        in_specs=[
            pl.BlockSpec((1, EMBED_DIM), lambda i: (0, 0)),
            pl.BlockSpec((MV_BLK, EMBED_DIM), lambda i: (i, 0)),
        ],
        out_specs=pl.BlockSpec((MV_BLK // 128, 128), lambda i: (i, 0)),
        out_shape=jax.ShapeDtypeStruct((P_ROWS, 128), jnp.float32),
    )(W, emb_table)


def _sc_body(p_hbm, sent_hbm, b_hbm, out_hbm,
             p_v, s_v, acc_v, sh, tmp_v, res_v, b_v):
    c = lax.axis_index("c")
    s = lax.axis_index("s")

    zero = jnp.zeros(16, jnp.float32)

    half = s // NPAIR
    chunk = s - half * NPAIR
    base_v = half * P_HALF
    pltpu.sync_copy(sent_hbm.at[c, chunk], s_v)
    pltpu.sync_copy(p_hbm.at[pl.ds(base_v, P_HALF)], p_v)

    base_vec = jnp.full(16, base_v, jnp.int32)
    lim_vec = jnp.full(16, P_HALF, jnp.int32)
    izero = jnp.zeros(16, jnp.int32)

    def step(r, acc):
        base = r * L_PER_CORE
        out = []
        for g in range(GRP):
            idx = s_v[pl.ds(base + 16 * g, 16)] - base_vec
            m = (idx >= izero) & (idx < lim_vec)
            val = plsc.load_gather(p_v, [jnp.where(m, idx, izero)])
            out.append(acc[g] + jnp.where(m, val, 0.0))
        return tuple(out)

    accs = lax.fori_loop(0, R_PER_SUB, step, (zero,) * GRP,
                         unroll=UNROLL)
    for g in range(GRP):
        acc_v[pl.ds(16 * g, 16)] = accs[g]
    pltpu.sync_copy(acc_v, sh.at[pl.ds(s * L_PER_CORE, L_PER_CORE)])

    plsc.subcore_barrier()

    @pl.when(s == 0)
    def _finish():
        pltpu.sync_copy(sh, tmp_v)
        pltpu.sync_copy(b_hbm, b_v)
        tot = [zero] * GRP
        for i in range(NS):
            for g in range(GRP):
                tot[g] = tot[g] + tmp_v[pl.ds(i * L_PER_CORE + 16 * g, 16)]
        bias = b_v[...]
        inv_b = jnp.float32(1.0 / BATCH)
        for g in range(GRP):
            sc = tot[g] * inv_b + bias
            res_v[pl.ds(16 * g, 16)] = 1.0 / (1.0 + jnp.exp(-sc))
        pltpu.sync_copy(res_v, out_hbm.at[c])


@functools.cache
def _get_sc_call():
    return functools.partial(
        pl.kernel,
        out_type=jax.ShapeDtypeStruct((NC, L_PER_CORE), jnp.float32),
        mesh=plsc.VectorSubcoreMesh(core_axis_name="c", subcore_axis_name="s",
                                    num_cores=NC, num_subcores=NS),
        compiler_params=pltpu.CompilerParams(needs_layout_passes=False),
        scratch_types=[
            pltpu.VMEM((P_HALF,), jnp.float32),
            pltpu.VMEM((R_PER_SUB * L_PER_CORE,), jnp.int32),
            pltpu.VMEM((L_PER_CORE,), jnp.float32),
            pltpu.VMEM_SHARED((NS * L_PER_CORE,), jnp.float32),
            pltpu.VMEM((NS * L_PER_CORE,), jnp.float32),
            pltpu.VMEM((L_PER_CORE,), jnp.float32),
            pltpu.VMEM((16,), jnp.float32),
        ],
    )(_sc_body)


def kernel(sentence, emb_table, W, b):
    p2 = _matvec(emb_table, W)
    p_flat = p2.reshape(P_FLAT)

    sent = sentence.astype(jnp.int32)
    sent = jnp.concatenate(
        [sent, jnp.zeros((BATCH, L_PAD - HIST), jnp.int32)], axis=1)
    sent_r = (sent.reshape(NPAIR, R_PER_SUB, NC, L_PER_CORE)
              .transpose(2, 0, 1, 3).reshape(NC, NPAIR, R_PER_SUB * L_PER_CORE))

    b16 = jnp.full((16,), b[0], jnp.float32)

    out = _get_sc_call()(p_flat, sent_r, b16)
    return out.reshape(L_PAD)[:HIST].reshape(HIST, 1)


# speedup vs baseline: 1.0607x; 1.0607x over previous
import functools

import jax
import jax.numpy as jnp
from jax import lax
from jax.experimental import pallas as pl
from jax.experimental.pallas import tpu as pltpu
from jax.experimental.pallas import tpu_sc as plsc

VOCAB = 100000
EMBED_DIM = 128
BATCH = 4096
HIST = 50

L_PAD = 64
NC, NS = 1, 16
ACT = 16
L_PER_CORE = L_PAD // NC
GRP = L_PER_CORE // 16
R_PER_SUB = BATCH // ACT
UNROLL = 8

MV_BLK = 16384
MV_GRID = (VOCAB + MV_BLK - 1) // MV_BLK
P_ROWS = MV_GRID * (MV_BLK // 128)
P_FLAT = P_ROWS * 128
P_SC = 106496


def _mv_body(w_ref, e_ref, o_ref):
    e = e_ref[...]
    w = w_ref[...][0]
    prod = e.reshape(MV_BLK // 128, 128, 128) * w
    o_ref[...] = jnp.sum(prod, axis=-1)


def _matvec(emb_table, W):
    return pl.pallas_call(
        _mv_body,
        grid=(MV_GRID,),
        in_specs=[
            pl.BlockSpec((1, EMBED_DIM), lambda i: (0, 0)),
            pl.BlockSpec((MV_BLK, EMBED_DIM), lambda i: (i, 0)),
        ],
        out_specs=pl.BlockSpec((MV_BLK // 128, 128), lambda i: (i, 0)),
        out_shape=jax.ShapeDtypeStruct((P_ROWS, 128), jnp.float32),
    )(W, emb_table)


def _sc_body(p_hbm, sent_hbm, b_hbm, out_hbm,
             p_v, s_v, acc_v, sh, tmp_v, res_v, b_v, sem_s, sem_p):
    c = lax.axis_index("c")
    s = lax.axis_index("s")

    zero = jnp.zeros((16,), jnp.float32)

    @pl.when(s < ACT)
    def _gather_phase():
        cp_s = pltpu.make_async_copy(sent_hbm.at[c, s], s_v, sem_s)
        cp_p = pltpu.make_async_copy(p_hbm.at[pl.ds(0, P_SC)], p_v, sem_p)
        cp_s.start()
        cp_p.start()
        cp_s.wait()
        cp_p.wait()

        def step(r, acc):
            base = r * L_PER_CORE
            out = []
            for g in range(GRP):
                idx = s_v[pl.ds(base + 16 * g, 16)]
                out.append(acc[g] + plsc.load_gather(p_v, [idx]))
            return tuple(out)

        accs = lax.fori_loop(0, R_PER_SUB, step, (zero,) * GRP,
                             unroll=UNROLL)
        for g in range(GRP):
            acc_v[pl.ds(16 * g, 16)] = accs[g]
        pltpu.sync_copy(acc_v, sh.at[pl.ds(s * L_PER_CORE, L_PER_CORE)])

    plsc.subcore_barrier()

    @pl.when(s == 0)
    def _finish():
        pltpu.sync_copy(sh, tmp_v)
        pltpu.sync_copy(b_hbm, b_v)
        tot = [zero] * GRP
        for i in range(ACT):
            for g in range(GRP):
                tot[g] = tot[g] + tmp_v[pl.ds(i * L_PER_CORE + 16 * g, 16)]
        bias = b_v[...]
        inv_b = jnp.float32(1.0 / BATCH)
        for g in range(GRP):
            sc = tot[g] * inv_b + bias
            res_v[pl.ds(16 * g, 16)] = 1.0 / (1.0 + jnp.exp(-sc))
        pltpu.sync_copy(res_v, out_hbm.at[c])


@functools.cache
def _get_sc_call():
    return functools.partial(
        pl.kernel,
        out_type=jax.ShapeDtypeStruct((NC, L_PER_CORE), jnp.float32),
        mesh=plsc.VectorSubcoreMesh(core_axis_name="c", subcore_axis_name="s",
                                    num_cores=NC, num_subcores=NS),
        compiler_params=pltpu.CompilerParams(needs_layout_passes=False),
        scratch_types=[
            pltpu.VMEM((P_SC,), jnp.float32),
            pltpu.VMEM((R_PER_SUB * L_PER_CORE,), jnp.int32),
            pltpu.VMEM((L_PER_CORE,), jnp.float32),
            pltpu.VMEM_SHARED((ACT * L_PER_CORE,), jnp.float32),
            pltpu.VMEM((ACT * L_PER_CORE,), jnp.float32),
            pltpu.VMEM((L_PER_CORE,), jnp.float32),
            pltpu.VMEM((16,), jnp.float32),
            pltpu.SemaphoreType.DMA,
            pltpu.SemaphoreType.DMA,
        ],
    )(_sc_body)


def kernel(sentence, emb_table, W, b):
    p2 = _matvec(emb_table, W)
    p_flat = p2.reshape(P_FLAT)

    sent = sentence.astype(jnp.int32)
    sent = jnp.concatenate(
        [sent, jnp.zeros((BATCH, L_PAD - HIST), jnp.int32)], axis=1)
    sent_r = (sent.reshape(ACT, R_PER_SUB, NC, L_PER_CORE)
              .transpose(2, 0, 1, 3).reshape(NC, ACT, R_PER_SUB * L_PER_CORE))

    b16 = jnp.full((16,), b[0], jnp.float32)

    out = _get_sc_call()(p_flat, sent_r, b16)
    return out.reshape(L_PAD)[:HIST].reshape(HIST, 1)


# speedup vs baseline: 1.0690x; 1.0078x over previous
import functools

import jax
import jax.numpy as jnp
from jax import lax
from jax.experimental import pallas as pl
from jax.experimental.pallas import tpu as pltpu
from jax.experimental.pallas import tpu_sc as plsc

VOCAB = 100000
EMBED_DIM = 128
BATCH = 4096
HIST = 50

L_PAD = 64
NC, NS = 1, 16
ACT = 16
L_PER_CORE = L_PAD // NC
GRP = L_PER_CORE // 16
R_PER_SUB = BATCH // ACT
UNROLL = 8

MV_BLK = 11264
MV_GRID = (VOCAB + MV_BLK - 1) // MV_BLK
P_ROWS = MV_GRID * (MV_BLK // 128)
P_FLAT = P_ROWS * 128
P_SC = P_FLAT


def _mv_body(w_ref, e_ref, o_ref):
    e = e_ref[...]
    w = w_ref[...][0]
    prod = e.reshape(MV_BLK // 128, 128, 128) * w
    o_ref[...] = jnp.sum(prod, axis=-1)


def _matvec(emb_table, W):
    return pl.pallas_call(
        _mv_body,
        grid=(MV_GRID,),
        in_specs=[
            pl.BlockSpec((1, EMBED_DIM), lambda i: (0, 0)),
            pl.BlockSpec((MV_BLK, EMBED_DIM), lambda i: (i, 0)),
        ],
        out_specs=pl.BlockSpec((MV_BLK // 128, 128), lambda i: (i, 0)),
        out_shape=jax.ShapeDtypeStruct((P_ROWS, 128), jnp.float32),
    )(W, emb_table)


def _sc_body(p_hbm, sent_hbm, b_hbm, out_hbm,
             p_v, s_v, acc_v, sh, tmp_v, res_v, b_v, sem_s, sem_p):
    c = lax.axis_index("c")
    s = lax.axis_index("s")

    zero = jnp.zeros((16,), jnp.float32)

    @pl.when(s < ACT)
    def _gather_phase():
        cp_s = pltpu.make_async_copy(sent_hbm.at[c, s], s_v, sem_s)
        cp_p = pltpu.make_async_copy(p_hbm.at[pl.ds(0, P_SC)], p_v, sem_p)
        cp_s.start()
        cp_p.start()
        cp_s.wait()
        cp_p.wait()

        def step(r, acc):
            base = r * L_PER_CORE
            out = []
            for g in range(GRP):
                idx = s_v[pl.ds(base + 16 * g, 16)]
                out.append(acc[g] + plsc.load_gather(p_v, [idx]))
            return tuple(out)

        accs = lax.fori_loop(0, R_PER_SUB, step, (zero,) * GRP,
                             unroll=UNROLL)
        for g in range(GRP):
            acc_v[pl.ds(16 * g, 16)] = accs[g]
        pltpu.sync_copy(acc_v, sh.at[pl.ds(s * L_PER_CORE, L_PER_CORE)])

    plsc.subcore_barrier()

    @pl.when(s == 0)
    def _finish():
        pltpu.sync_copy(sh, tmp_v)
        pltpu.sync_copy(b_hbm, b_v)
        tot = [zero] * GRP
        for i in range(ACT):
            for g in range(GRP):
                tot[g] = tot[g] + tmp_v[pl.ds(i * L_PER_CORE + 16 * g, 16)]
        bias = b_v[...]
        inv_b = jnp.float32(1.0 / BATCH)
        for g in range(GRP):
            sc = tot[g] * inv_b + bias
            res_v[pl.ds(16 * g, 16)] = 1.0 / (1.0 + jnp.exp(-sc))
        pltpu.sync_copy(res_v, out_hbm.at[c])


@functools.cache
def _get_sc_call():
    return functools.partial(
        pl.kernel,
        out_type=jax.ShapeDtypeStruct((NC, L_PER_CORE), jnp.float32),
        mesh=plsc.VectorSubcoreMesh(core_axis_name="c", subcore_axis_name="s",
                                    num_cores=NC, num_subcores=NS),
        compiler_params=pltpu.CompilerParams(
            needs_layout_passes=False,
            disable_bounds_checks=True,
            disable_semaphore_checks=True,
        ),
        scratch_types=[
            pltpu.VMEM((P_SC,), jnp.float32),
            pltpu.VMEM((R_PER_SUB * L_PER_CORE,), jnp.int32),
            pltpu.VMEM((L_PER_CORE,), jnp.float32),
            pltpu.VMEM_SHARED((ACT * L_PER_CORE,), jnp.float32),
            pltpu.VMEM((ACT * L_PER_CORE,), jnp.float32),
            pltpu.VMEM((L_PER_CORE,), jnp.float32),
            pltpu.VMEM((16,), jnp.float32),
            pltpu.SemaphoreType.DMA,
            pltpu.SemaphoreType.DMA,
        ],
    )(_sc_body)


def kernel(sentence, emb_table, W, b):
    p2 = _matvec(emb_table, W)
    p_flat = p2.reshape(P_FLAT)

    sent = sentence.astype(jnp.int32)
    sent = jnp.concatenate(
        [sent, jnp.zeros((BATCH, L_PAD - HIST), jnp.int32)], axis=1)
    sent_r = (sent.reshape(ACT, R_PER_SUB, NC, L_PER_CORE)
              .transpose(2, 0, 1, 3).reshape(NC, ACT, R_PER_SUB * L_PER_CORE))

    b16 = jnp.full((16,), b[0], jnp.float32)

    out = _get_sc_call()(p_flat, sent_r, b16)
    return out.reshape(L_PAD)[:HIST].reshape(HIST, 1)


# speedup vs baseline: 1.0708x; 1.0017x over previous
import functools

import jax
import jax.numpy as jnp
from jax import lax
from jax.experimental import pallas as pl
from jax.experimental.pallas import tpu as pltpu
from jax.experimental.pallas import tpu_sc as plsc

VOCAB = 100000
EMBED_DIM = 128
BATCH = 4096
HIST = 50

L_PAD = 64
NC, NS = 1, 16
ACT = 16
L_PER_CORE = L_PAD // NC
GRP = L_PER_CORE // 16
R_PER_SUB = BATCH // ACT
UNROLL = 8

MV_BLK = 11264
MV_GRID = (VOCAB + MV_BLK - 1) // MV_BLK
P_ROWS = MV_GRID * (MV_BLK // 128)
P_FLAT = P_ROWS * 128
P_SC = P_FLAT


def _mv_body(w_ref, e_ref, o_ref):
    e = e_ref[...]
    w = w_ref[...][0]
    prod = e.reshape(MV_BLK // 128, 128, 128) * w
    o_ref[...] = jnp.sum(prod, axis=-1)


def _matvec(emb_table, W):
    return pl.pallas_call(
        _mv_body,
        grid=(MV_GRID,),
        in_specs=[
            pl.BlockSpec((1, EMBED_DIM), lambda i: (0, 0)),
            pl.BlockSpec((MV_BLK, EMBED_DIM), lambda i: (i, 0)),
        ],
        out_specs=pl.BlockSpec((MV_BLK // 128, 128), lambda i: (i, 0)),
        out_shape=jax.ShapeDtypeStruct((P_ROWS, 128), jnp.float32),
        compiler_params=pltpu.CompilerParams(skip_device_barrier=True),
    )(W, emb_table)


def _sc_body(p_hbm, sent_hbm, b_hbm, out_hbm,
             p_v, s_v, acc_v, sh, tmp_v, res_v, b_v, sem_s, sem_p):
    c = lax.axis_index("c")
    s = lax.axis_index("s")

    zero = jnp.zeros((16,), jnp.float32)

    @pl.when(s < ACT)
    def _gather_phase():
        cp_s = pltpu.make_async_copy(sent_hbm.at[c, s], s_v, sem_s)
        cp_p = pltpu.make_async_copy(p_hbm.at[pl.ds(0, P_SC)], p_v, sem_p)
        cp_s.start()
        cp_p.start()
        cp_s.wait()
        cp_p.wait()

        def step(r, acc):
            base = r * L_PER_CORE
            out = []
            for g in range(GRP):
                idx = s_v[pl.ds(base + 16 * g, 16)]
                out.append(acc[g] + plsc.load_gather(p_v, [idx]))
            return tuple(out)

        accs = lax.fori_loop(0, R_PER_SUB, step, (zero,) * GRP,
                             unroll=UNROLL)
        for g in range(GRP):
            acc_v[pl.ds(16 * g, 16)] = accs[g]
        pltpu.sync_copy(acc_v, sh.at[pl.ds(s * L_PER_CORE, L_PER_CORE)])

    plsc.subcore_barrier()

    @pl.when(s == 0)
    def _finish():
        pltpu.sync_copy(sh, tmp_v)
        pltpu.sync_copy(b_hbm, b_v)
        tot = [zero] * GRP
        for i in range(ACT):
            for g in range(GRP):
                tot[g] = tot[g] + tmp_v[pl.ds(i * L_PER_CORE + 16 * g, 16)]
        bias = b_v[...]
        inv_b = jnp.float32(1.0 / BATCH)
        for g in range(GRP):
            sc = tot[g] * inv_b + bias
            res_v[pl.ds(16 * g, 16)] = 1.0 / (1.0 + jnp.exp(-sc))
        pltpu.sync_copy(res_v, out_hbm.at[c])


@functools.cache
def _get_sc_call():
    return functools.partial(
        pl.kernel,
        out_type=jax.ShapeDtypeStruct((NC, L_PER_CORE), jnp.float32),
        mesh=plsc.VectorSubcoreMesh(core_axis_name="c", subcore_axis_name="s",
                                    num_cores=NC, num_subcores=NS),
        compiler_params=pltpu.CompilerParams(
            needs_layout_passes=False,
            disable_bounds_checks=True,
            disable_semaphore_checks=True,
            skip_device_barrier=True,
        ),
        scratch_types=[
            pltpu.VMEM((P_SC,), jnp.float32),
            pltpu.VMEM((R_PER_SUB * L_PER_CORE,), jnp.int32),
            pltpu.VMEM((L_PER_CORE,), jnp.float32),
            pltpu.VMEM_SHARED((ACT * L_PER_CORE,), jnp.float32),
            pltpu.VMEM((ACT * L_PER_CORE,), jnp.float32),
            pltpu.VMEM((L_PER_CORE,), jnp.float32),
            pltpu.VMEM((16,), jnp.float32),
            pltpu.SemaphoreType.DMA,
            pltpu.SemaphoreType.DMA,
        ],
    )(_sc_body)


def kernel(sentence, emb_table, W, b):
    p2 = _matvec(emb_table, W)
    p_flat = p2.reshape(P_FLAT)

    sent = sentence.astype(jnp.int32)
    sent = jnp.concatenate(
        [sent, jnp.zeros((BATCH, L_PAD - HIST), jnp.int32)], axis=1)
    sent_r = (sent.reshape(ACT, R_PER_SUB, NC, L_PER_CORE)
              .transpose(2, 0, 1, 3).reshape(NC, ACT, R_PER_SUB * L_PER_CORE))

    b16 = jnp.full((16,), b[0], jnp.float32)

    out = _get_sc_call()(p_flat, sent_r, b16)
    return out.reshape(L_PAD)[:HIST].reshape(HIST, 1)
